# baseline (speedup 1.0000x reference)
import functools
import numpy as np
import jax
import jax.numpy as jnp
from jax import lax
from jax.experimental import pallas as pl
from jax.experimental.pallas import tpu as pltpu
from jax.experimental.pallas import tpu_sc as plsc

_N_ATOMS = 5000
_N_GRID = 48
_GRID = 0.5
_VDW = np.array([1.7, 1.55, 1.52, 1.8, 1.95, 1.8, 1.4, 1.7], dtype=np.float32)
_NPAD = 5008
_NF = 13
_SLAB = 12
_L = 16
_YS = _N_GRID
_XS = _N_GRID * _N_GRID
_GSZ = _SLAB * _XS
_WLSZ = _NPAD + 2 * _L


def _splat_f32(bits_scalar):
    return plsc.bitcast(jnp.full((_L,), bits_scalar, dtype=jnp.int32),
                        jnp.float32)


def _sc_body(atoms_hbm, out_hbm, atoms_v, grid_v, wl_v, dxb_v, fyb_v):
    cid = lax.axis_index("c")
    sid = lax.axis_index("s")
    wid = sid * 2 + cid
    ch = wid // 4
    slab = wid % 4
    lo = slab * _SLAB
    hi = lo + _SLAB

    pltpu.sync_copy(atoms_hbm, atoms_v)

    zero = jnp.zeros((_L,), jnp.float32)

    @plsc.parallel_loop(0, _GSZ, step=_L, unroll=8)
    def zbody(k):
        grid_v[pl.ds(k, _L)] = zero

    iota = lax.broadcasted_iota(jnp.int32, (_L,), 0)

    def sbody(g, cnt):
        s = g * _L
        t = atoms_v[pl.ds(s, _L)]
        mnx = atoms_v[pl.ds(1 * _NPAD + s, _L)]
        mxx = atoms_v[pl.ds(2 * _NPAD + s, _L)]
        m = ((t == ch) & (mnx < hi)) & (mxx > lo)
        pos_inc = plsc.cumsum(m.astype(jnp.int32))
        pos = jnp.where(m, cnt + pos_inc - 1, jnp.int32(_NPAD + _L))
        plsc.store_scatter(wl_v, [pos], s + iota)
        return cnt + pos_inc[_L - 1]

    cnt = lax.fori_loop(0, _NPAD // _L, sbody, jnp.int32(0))

    three_over_e = jnp.full((_L,), 3.0 / np.e, dtype=jnp.float32)
    magic = jnp.int32(0x5F3759DF)

    def abody(j, carry):
        ai = wl_v[pl.ds(j, _L)][0]
        fv = plsc.load_gather(
            atoms_v, [jnp.minimum(iota, _NF - 1) * _NPAD + ai])
        mnx = fv[1]
        mxx = fv[2]
        mny = fv[3]
        mxy = fv[4]
        mnz = fv[5]
        mxz = fv[6]
        cxv = _splat_f32(fv[7])
        cyv = _splat_f32(fv[8])
        czv = _splat_f32(fv[9])
        r2v = _splat_f32(fv[10])
        c1v = _splat_f32(fv[11])
        a2v = _splat_f32(fv[12])
        r2q = r2v * 2.25

        zb = jnp.minimum(mnz, _N_GRID - _L)
        izv = zb + iota
        zm = (izv >= mnz) & (izv < mxz)
        fzv = izv.astype(jnp.float32) * _GRID - czv
        dz2 = jnp.where(zm, fzv * fzv, 1e9)

        x0 = jnp.maximum(mnx, lo)
        x1 = jnp.minimum(mxx, hi)
        nx = x1 - x0
        yb = jnp.minimum(mny, _N_GRID - 14)

        @plsc.parallel_loop(0, nx, unroll=2)
        def pxbody(t):
            fxv = (jnp.full((_L,), x0 + t).astype(jnp.float32) * _GRID
                   - cxv)
            dxb_v[pl.ds(t * _L, _L)] = fxv * fxv + dz2

        @plsc.parallel_loop(0, 14, unroll=7)
        def pybody(u):
            iy = yb + u
            yok = (iy >= mny) & (iy < mxy)
            fyv = (jnp.full((_L,), iy).astype(jnp.float32) * _GRID
                   - cyv)
            fyb_v[pl.ds(u * _L, _L)] = jnp.where(
                jnp.full((_L,), yok), fyv * fyv, 1e9)

        base = (x0 - lo) * _XS + yb * _YS + zb

        @plsc.parallel_loop(0, nx, unroll=1)
        def xrow(t):
            dx2 = dxb_v[pl.ds(t * _L, _L)]
            baset = base + t * _XS

            @plsc.parallel_loop(0, 14, unroll=14)
            def rbody(u):
                d2 = dx2 + fyb_v[pl.ds(u * _L, _L)]
                uu = plsc.bitcast(d2, jnp.int32)
                yv = plsc.bitcast(magic - lax.shift_right_logical(uu, 1),
                                  jnp.float32)
                yv = yv * (1.5 - 0.5 * d2 * yv * yv)
                d = d2 * yv
                f1 = jnp.exp(c1v * d2)
                t2 = a2v * d - three_over_e
                f2 = t2 * t2
                val = jnp.where(d2 < r2v, f1,
                                jnp.where(d2 < r2q, f2, zero))
                plsc.addupdate(
                    grid_v.at[pl.ds(baset + u * _YS, _L)], val)

        return carry

    lax.fori_loop(0, cnt, abody, 0)

    pltpu.sync_copy(grid_v, out_hbm.at[ch, slab])


def kernel(coords, atom_types):
    r = jnp.asarray(_VDW)[atom_types]
    b = 1.5 * r
    minf = (coords - b[:, None]) / _GRID
    maxf = 2.0 + (coords + b[:, None]) / _GRID
    minp = jnp.clip(jnp.trunc(minf).astype(jnp.int32), 0, _N_GRID)
    maxp = jnp.clip(jnp.trunc(maxf).astype(jnp.int32), 0, _N_GRID)
    r2 = r * r
    c1 = -2.0 / r2
    a2 = 2.0 / (float(np.e) * r)

    pad = _NPAD - _N_ATOMS
    zpad_i = jnp.zeros((pad,), jnp.int32)

    def fbits(x):
        return jnp.concatenate(
            [lax.bitcast_convert_type(x, jnp.int32), zpad_i])

    def ipad(x):
        return jnp.concatenate([x, zpad_i])

    atoms = jnp.concatenate([
        jnp.concatenate([atom_types.astype(jnp.int32),
                         jnp.full((pad,), -1, jnp.int32)]),
        ipad(minp[:, 0]), ipad(maxp[:, 0]),
        ipad(minp[:, 1]), ipad(maxp[:, 1]),
        ipad(minp[:, 2]), ipad(maxp[:, 2]),
        fbits(coords[:, 0]), fbits(coords[:, 1]), fbits(coords[:, 2]),
        fbits(r2), fbits(c1), fbits(a2),
    ])

    mesh = plsc.VectorSubcoreMesh(core_axis_name="c", subcore_axis_name="s")
    f = functools.partial(
        pl.kernel,
        out_type=jax.ShapeDtypeStruct((8, 4, _GSZ), jnp.float32),
        mesh=mesh,
        scratch_types=[
            pltpu.VMEM((_NF * _NPAD,), jnp.int32),
            pltpu.VMEM((_GSZ,), jnp.float32),
            pltpu.VMEM((_WLSZ,), jnp.int32),
            pltpu.VMEM(((_SLAB + 2) * _L,), jnp.float32),
            pltpu.VMEM((_L * _L,), jnp.float32),
        ],
        compiler_params=pltpu.CompilerParams(needs_layout_passes=False),
    )(_sc_body)
    res = f(atoms)
    return res.reshape(8, 4, _SLAB, _N_GRID, _N_GRID).reshape(
        8, _N_GRID, _N_GRID, _N_GRID)


# speedup vs baseline: 1.0713x; 1.0713x over previous
import functools
import numpy as np
import jax
import jax.numpy as jnp
from jax import lax
from jax.experimental import pallas as pl
from jax.experimental.pallas import tpu as pltpu
from jax.experimental.pallas import tpu_sc as plsc

_N_ATOMS = 5000
_N_GRID = 48
_GRID = 0.5
_VDW = np.array([1.7, 1.55, 1.52, 1.8, 1.95, 1.8, 1.4, 1.7], dtype=np.float32)
_NPAD = 5008
_NF = 13
_SLAB = 12
_L = 16
_YS = _N_GRID
_XS = _N_GRID * _N_GRID
_GSZ = _SLAB * _XS
_WLSZ = _NPAD + 2 * _L


def _splat_f32(bits_scalar):
    return plsc.bitcast(jnp.full((_L,), bits_scalar, dtype=jnp.int32),
                        jnp.float32)


def _sc_body(atoms_hbm, out_hbm, atoms_v, grid_v, wl_v, dxb_v, fyb_v):
    cid = lax.axis_index("c")
    sid = lax.axis_index("s")
    wid = sid * 2 + cid
    ch = wid // 4
    slab = wid % 4
    lo = slab * _SLAB
    hi = lo + _SLAB

    pltpu.sync_copy(atoms_hbm, atoms_v)

    zero = jnp.zeros((_L,), jnp.float32)

    @plsc.parallel_loop(0, _GSZ, step=_L, unroll=8)
    def zbody(k):
        grid_v[pl.ds(k, _L)] = zero

    iota = lax.broadcasted_iota(jnp.int32, (_L,), 0)

    def sbody(g, cnt):
        s = g * _L
        t = atoms_v[pl.ds(s, _L)]
        mnx = atoms_v[pl.ds(1 * _NPAD + s, _L)]
        mxx = atoms_v[pl.ds(2 * _NPAD + s, _L)]
        m = ((t == ch) & (mnx < hi)) & (mxx > lo)
        pos_inc = plsc.cumsum(m.astype(jnp.int32))
        pos = jnp.where(m, cnt + pos_inc - 1, jnp.int32(_NPAD + _L))
        plsc.store_scatter(wl_v, [pos], s + iota)
        return cnt + pos_inc[_L - 1]

    cnt = lax.fori_loop(0, _NPAD // _L, sbody, jnp.int32(0))

    three_over_e = jnp.full((_L,), 3.0 / np.e, dtype=jnp.float32)
    magic = jnp.int32(0x5F3759DF)

    def abody(j, carry):
        ai = wl_v[pl.ds(j, _L)][0]
        fv = plsc.load_gather(
            atoms_v, [jnp.minimum(iota, _NF - 1) * _NPAD + ai])
        mnx = fv[1]
        mxx = fv[2]
        mny = fv[3]
        mxy = fv[4]
        mnz = fv[5]
        mxz = fv[6]
        cxv = _splat_f32(fv[7])
        cyv = _splat_f32(fv[8])
        czv = _splat_f32(fv[9])
        r2v = _splat_f32(fv[10])
        c1v = _splat_f32(fv[11])
        a2v = _splat_f32(fv[12])
        r2q = r2v * 2.25

        zb = jnp.minimum(mnz, _N_GRID - _L)
        izv = zb + iota
        zm = (izv >= mnz) & (izv < mxz)
        fzv = izv.astype(jnp.float32) * _GRID - czv
        dz2 = jnp.where(zm, fzv * fzv, 1e9)

        x0 = jnp.maximum(mnx, lo)
        x1 = jnp.minimum(mxx, hi)
        nx = x1 - x0
        yb = jnp.minimum(mny, _N_GRID - 14)

        @plsc.parallel_loop(0, nx, unroll=2)
        def pxbody(t):
            fxv = (jnp.full((_L,), x0 + t).astype(jnp.float32) * _GRID
                   - cxv)
            dxb_v[pl.ds(t * _L, _L)] = fxv * fxv + dz2

        @plsc.parallel_loop(0, 14, unroll=7)
        def pybody(u):
            iy = yb + u
            yok = (iy >= mny) & (iy < mxy)
            fyv = (jnp.full((_L,), iy).astype(jnp.float32) * _GRID
                   - cyv)
            fyb_v[pl.ds(u * _L, _L)] = jnp.where(
                jnp.full((_L,), yok), fyv * fyv, 1e9)

        base = (x0 - lo) * _XS + yb * _YS + zb

        def xrow(t, carry_x):
            dx2 = dxb_v[pl.ds(t * _L, _L)]
            baset = base + t * _XS

            @plsc.parallel_loop(0, 14, unroll=14)
            def rbody(u):
                d2 = dx2 + fyb_v[pl.ds(u * _L, _L)]
                uu = plsc.bitcast(d2, jnp.int32)
                yv = plsc.bitcast(magic - lax.shift_right_logical(uu, 1),
                                  jnp.float32)
                yv = yv * (1.5 - 0.5 * d2 * yv * yv)
                d = d2 * yv
                f1 = jnp.exp(c1v * d2)
                t2 = a2v * d - three_over_e
                f2 = t2 * t2
                val = jnp.where(d2 < r2v, f1,
                                jnp.where(d2 < r2q, f2, zero))
                plsc.addupdate(
                    grid_v.at[pl.ds(baset + u * _YS, _L)], val)

            return carry_x

        lax.fori_loop(0, nx, xrow, 0)
        return carry

    lax.fori_loop(0, cnt, abody, 0)

    pltpu.sync_copy(grid_v, out_hbm.at[ch, slab])


def kernel(coords, atom_types):
    r = jnp.asarray(_VDW)[atom_types]
    b = 1.5 * r
    minf = (coords - b[:, None]) / _GRID
    maxf = 2.0 + (coords + b[:, None]) / _GRID
    minp = jnp.clip(jnp.trunc(minf).astype(jnp.int32), 0, _N_GRID)
    maxp = jnp.clip(jnp.trunc(maxf).astype(jnp.int32), 0, _N_GRID)
    r2 = r * r
    c1 = -2.0 / r2
    a2 = 2.0 / (float(np.e) * r)

    pad = _NPAD - _N_ATOMS
    zpad_i = jnp.zeros((pad,), jnp.int32)

    def fbits(x):
        return jnp.concatenate(
            [lax.bitcast_convert_type(x, jnp.int32), zpad_i])

    def ipad(x):
        return jnp.concatenate([x, zpad_i])

    atoms = jnp.concatenate([
        jnp.concatenate([atom_types.astype(jnp.int32),
                         jnp.full((pad,), -1, jnp.int32)]),
        ipad(minp[:, 0]), ipad(maxp[:, 0]),
        ipad(minp[:, 1]), ipad(maxp[:, 1]),
        ipad(minp[:, 2]), ipad(maxp[:, 2]),
        fbits(coords[:, 0]), fbits(coords[:, 1]), fbits(coords[:, 2]),
        fbits(r2), fbits(c1), fbits(a2),
    ])

    mesh = plsc.VectorSubcoreMesh(core_axis_name="c", subcore_axis_name="s")
    f = functools.partial(
        pl.kernel,
        out_type=jax.ShapeDtypeStruct((8, 4, _GSZ), jnp.float32),
        mesh=mesh,
        scratch_types=[
            pltpu.VMEM((_NF * _NPAD,), jnp.int32),
            pltpu.VMEM((_GSZ,), jnp.float32),
            pltpu.VMEM((_WLSZ,), jnp.int32),
            pltpu.VMEM(((_SLAB + 2) * _L,), jnp.float32),
            pltpu.VMEM((_L * _L,), jnp.float32),
        ],
        compiler_params=pltpu.CompilerParams(needs_layout_passes=False),
    )(_sc_body)
    res = f(atoms)
    return res.reshape(8, 4, _SLAB, _N_GRID, _N_GRID).reshape(
        8, _N_GRID, _N_GRID, _N_GRID)


# speedup vs baseline: 1.1026x; 1.0292x over previous
import functools
import numpy as np
import jax
import jax.numpy as jnp
from jax import lax
from jax.experimental import pallas as pl
from jax.experimental.pallas import tpu as pltpu
from jax.experimental.pallas import tpu_sc as plsc

_N_ATOMS = 5000
_N_GRID = 48
_GRID = 0.5
_VDW = np.array([1.7, 1.55, 1.52, 1.8, 1.95, 1.8, 1.4, 1.7], dtype=np.float32)
_NPAD = 5008
_NF = 13
_SLAB = 12
_L = 16
_YS = _N_GRID
_XS = _N_GRID * _N_GRID
_GSZ = _SLAB * _XS
_WLSZ = _NPAD + 2 * _L


def _splat_f32(bits_scalar):
    return plsc.bitcast(jnp.full((_L,), bits_scalar, dtype=jnp.int32),
                        jnp.float32)


def _sc_body(atoms_hbm, out_hbm, atoms_v, grid_v, wl_v, dxb_v, fyb_v):
    cid = lax.axis_index("c")
    sid = lax.axis_index("s")
    wid = sid * 2 + cid
    ch = wid // 4
    slab = wid % 4
    lo = slab * _SLAB
    hi = lo + _SLAB

    pltpu.sync_copy(atoms_hbm, atoms_v)

    zero = jnp.zeros((_L,), jnp.float32)

    @plsc.parallel_loop(0, _GSZ, step=_L, unroll=8)
    def zbody(k):
        grid_v[pl.ds(k, _L)] = zero

    iota = lax.broadcasted_iota(jnp.int32, (_L,), 0)

    def sbody(g, cnt):
        s = g * _L
        t = atoms_v[pl.ds(s, _L)]
        mnx = atoms_v[pl.ds(1 * _NPAD + s, _L)]
        mxx = atoms_v[pl.ds(2 * _NPAD + s, _L)]
        m = ((t == ch) & (mnx < hi)) & (mxx > lo)
        pos_inc = plsc.cumsum(m.astype(jnp.int32))
        pos = jnp.where(m, cnt + pos_inc - 1, jnp.int32(_NPAD + _L))
        plsc.store_scatter(wl_v, [pos], s + iota)
        return cnt + pos_inc[_L - 1]

    cnt = lax.fori_loop(0, _NPAD // _L, sbody, jnp.int32(0))

    three_over_e = jnp.full((_L,), 3.0 / np.e, dtype=jnp.float32)
    magic = jnp.int32(0x5F3759DF)

    def abody(j, carry):
        ai = wl_v[pl.ds(j, _L)][0]
        fv = plsc.load_gather(
            atoms_v, [jnp.minimum(iota, _NF - 1) * _NPAD + ai])
        mnx = fv[1]
        mxx = fv[2]
        mny = fv[3]
        mxy = fv[4]
        mnz = fv[5]
        mxz = fv[6]
        cxv = _splat_f32(fv[7])
        cyv = _splat_f32(fv[8])
        czv = _splat_f32(fv[9])
        r2v = _splat_f32(fv[10])
        c1v = _splat_f32(fv[11])
        a2v = _splat_f32(fv[12])
        r2q = r2v * 2.25

        zb = jnp.minimum(mnz, _N_GRID - _L)
        izv = zb + iota
        zm = (izv >= mnz) & (izv < mxz)
        fzv = izv.astype(jnp.float32) * _GRID - czv
        dz2 = jnp.where(zm, fzv * fzv, 1e9)

        x0 = jnp.maximum(mnx, lo)
        x1 = jnp.minimum(mxx, hi)
        nx = x1 - x0
        yb = jnp.minimum(mny, _N_GRID - 14)

        @plsc.parallel_loop(0, nx, unroll=2)
        def pxbody(t):
            fxv = (jnp.full((_L,), x0 + t).astype(jnp.float32) * _GRID
                   - cxv)
            dxb_v[pl.ds(t * _L, _L)] = fxv * fxv + dz2

        @plsc.parallel_loop(0, 14, unroll=2)
        def pybody(u):
            iy = yb + u
            yok = (iy >= mny) & (iy < mxy)
            fyv = (jnp.full((_L,), iy).astype(jnp.float32) * _GRID
                   - cyv)
            fyb_v[pl.ds(u * _L, _L)] = jnp.where(
                jnp.full((_L,), yok), fyv * fyv, 1e9)

        base = (x0 - lo) * _XS + yb * _YS + zb

        def xrow(t, carry_x):
            dx2 = dxb_v[pl.ds(t * _L, _L)]
            baset = base + t * _XS

            @plsc.parallel_loop(0, 14, unroll=14)
            def rbody(u):
                d2 = dx2 + fyb_v[pl.ds(u * _L, _L)]
                uu = plsc.bitcast(d2, jnp.int32)
                yv = plsc.bitcast(magic - lax.shift_right_logical(uu, 1),
                                  jnp.float32)
                yv = yv * (1.5 - 0.5 * d2 * yv * yv)
                d = d2 * yv
                f1 = jnp.exp(c1v * d2)
                t2 = a2v * d - three_over_e
                f2 = t2 * t2
                val = jnp.where(d2 < r2v, f1,
                                jnp.where(d2 < r2q, f2, zero))
                plsc.addupdate(
                    grid_v.at[pl.ds(baset + u * _YS, _L)], val)

            return carry_x

        lax.fori_loop(0, nx, xrow, 0)
        return carry

    lax.fori_loop(0, cnt, abody, 0)

    pltpu.sync_copy(grid_v, out_hbm.at[ch, slab])


def kernel(coords, atom_types):
    r = jnp.asarray(_VDW)[atom_types]
    b = 1.5 * r
    minf = (coords - b[:, None]) / _GRID
    maxf = 2.0 + (coords + b[:, None]) / _GRID
    minp = jnp.clip(jnp.trunc(minf).astype(jnp.int32), 0, _N_GRID)
    maxp = jnp.clip(jnp.trunc(maxf).astype(jnp.int32), 0, _N_GRID)
    r2 = r * r
    c1 = -2.0 / r2
    a2 = 2.0 / (float(np.e) * r)

    pad = _NPAD - _N_ATOMS
    zpad_i = jnp.zeros((pad,), jnp.int32)

    def fbits(x):
        return jnp.concatenate(
            [lax.bitcast_convert_type(x, jnp.int32), zpad_i])

    def ipad(x):
        return jnp.concatenate([x, zpad_i])

    atoms = jnp.concatenate([
        jnp.concatenate([atom_types.astype(jnp.int32),
                         jnp.full((pad,), -1, jnp.int32)]),
        ipad(minp[:, 0]), ipad(maxp[:, 0]),
        ipad(minp[:, 1]), ipad(maxp[:, 1]),
        ipad(minp[:, 2]), ipad(maxp[:, 2]),
        fbits(coords[:, 0]), fbits(coords[:, 1]), fbits(coords[:, 2]),
        fbits(r2), fbits(c1), fbits(a2),
    ])

    mesh = plsc.VectorSubcoreMesh(core_axis_name="c", subcore_axis_name="s")
    f = functools.partial(
        pl.kernel,
        out_type=jax.ShapeDtypeStruct((8, 4, _GSZ), jnp.float32),
        mesh=mesh,
        scratch_types=[
            pltpu.VMEM((_NF * _NPAD,), jnp.int32),
            pltpu.VMEM((_GSZ,), jnp.float32),
            pltpu.VMEM((_WLSZ,), jnp.int32),
            pltpu.VMEM(((_SLAB + 2) * _L,), jnp.float32),
            pltpu.VMEM((_L * _L,), jnp.float32),
        ],
        compiler_params=pltpu.CompilerParams(needs_layout_passes=False),
    )(_sc_body)
    res = f(atoms)
    return res.reshape(8, 4, _SLAB, _N_GRID, _N_GRID).reshape(
        8, _N_GRID, _N_GRID, _N_GRID)


# speedup vs baseline: 1.1229x; 1.0184x over previous
import functools
import numpy as np
import jax
import jax.numpy as jnp
from jax import lax
from jax.experimental import pallas as pl
from jax.experimental.pallas import tpu as pltpu
from jax.experimental.pallas import tpu_sc as plsc

_N_ATOMS = 5000
_N_GRID = 48
_GRID = 0.5
_VDW = np.array([1.7, 1.55, 1.52, 1.8, 1.95, 1.8, 1.4, 1.7], dtype=np.float32)
_NPAD = 5008
_NF = 13
_SLAB = 12
_L = 16
_YS = _N_GRID
_XS = _N_GRID * _N_GRID
_GSZ = _SLAB * _XS
_WLSZ = _NPAD + 2 * _L


def _splat_f32(bits_scalar):
    return plsc.bitcast(jnp.full((_L,), bits_scalar, dtype=jnp.int32),
                        jnp.float32)


def _sc_body(atoms_hbm, out_hbm, atoms_v, grid_v, wl_v, dxb_v, fyb_v):
    cid = lax.axis_index("c")
    sid = lax.axis_index("s")
    wid = sid * 2 + cid
    ch = wid // 4
    slab = wid % 4
    lo = slab * _SLAB
    hi = lo + _SLAB

    pltpu.sync_copy(atoms_hbm, atoms_v)

    zero = jnp.zeros((_L,), jnp.float32)

    @plsc.parallel_loop(0, _GSZ, step=_L, unroll=8)
    def zbody(k):
        grid_v[pl.ds(k, _L)] = zero

    iota = lax.broadcasted_iota(jnp.int32, (_L,), 0)

    def sbody(g, cnt):
        s = g * _L
        t = atoms_v[pl.ds(s, _L)]
        mnx = atoms_v[pl.ds(1 * _NPAD + s, _L)]
        mxx = atoms_v[pl.ds(2 * _NPAD + s, _L)]
        m = ((t == ch) & (mnx < hi)) & (mxx > lo)
        pos_inc = plsc.cumsum(m.astype(jnp.int32))
        pos = jnp.where(m, cnt + pos_inc - 1, jnp.int32(_NPAD + _L))
        plsc.store_scatter(wl_v, [pos], s + iota)
        return cnt + pos_inc[_L - 1]

    cnt = lax.fori_loop(0, _NPAD // _L, sbody, jnp.int32(0))

    three_over_e = jnp.full((_L,), 3.0 / np.e, dtype=jnp.float32)
    magic = jnp.int32(0x5F3759DF)

    def abody(j, carry):
        ai = wl_v[pl.ds(j, _L)][0]
        fv = plsc.load_gather(
            atoms_v, [jnp.minimum(iota, _NF - 1) * _NPAD + ai])
        mnx = fv[1]
        mxx = fv[2]
        mny = fv[3]
        mxy = fv[4]
        mnz = fv[5]
        mxz = fv[6]
        cxv = _splat_f32(fv[7])
        cyv = _splat_f32(fv[8])
        czv = _splat_f32(fv[9])
        r2v = _splat_f32(fv[10])
        c1v = _splat_f32(fv[11])
        a2v = _splat_f32(fv[12])
        r2q = r2v * 2.25

        zb = jnp.minimum(mnz, _N_GRID - _L)
        izv = zb + iota
        zm = (izv >= mnz) & (izv < mxz)
        fzv = izv.astype(jnp.float32) * _GRID - czv
        dz2 = jnp.where(zm, fzv * fzv, 1e9)

        x0 = jnp.maximum(mnx, lo)
        x1 = jnp.minimum(mxx, hi)
        nx = x1 - x0
        yb = jnp.minimum(mny, _N_GRID - 14)

        @plsc.parallel_loop(0, 14, unroll=2)
        def pybody(u):
            iy = yb + u
            yok = (iy >= mny) & (iy < mxy)
            fyv = (jnp.full((_L,), iy).astype(jnp.float32) * _GRID
                   - cyv)
            fyb_v[pl.ds(u * _L, _L)] = jnp.where(
                jnp.full((_L,), yok), fyv * fyv, 1e9)

        base = (x0 - lo) * _XS + yb * _YS + zb

        def xrow(t, carry_x):
            fxv = (jnp.full((_L,), x0 + t).astype(jnp.float32) * _GRID
                   - cxv)
            dx2 = fxv * fxv + dz2
            baset = base + t * _XS

            @plsc.parallel_loop(0, 14, unroll=14)
            def rbody(u):
                d2 = dx2 + fyb_v[pl.ds(u * _L, _L)]
                uu = plsc.bitcast(d2, jnp.int32)
                yv = plsc.bitcast(magic - lax.shift_right_logical(uu, 1),
                                  jnp.float32)
                yv = yv * (1.5 - 0.5 * d2 * yv * yv)
                d = d2 * yv
                f1 = jnp.exp(c1v * d2)
                t2 = a2v * d - three_over_e
                f2 = t2 * t2
                val = jnp.where(d2 < r2v, f1,
                                jnp.where(d2 < r2q, f2, zero))
                plsc.addupdate(
                    grid_v.at[pl.ds(baset + u * _YS, _L)], val)

            return carry_x

        lax.fori_loop(0, nx, xrow, 0)
        return carry

    lax.fori_loop(0, cnt, abody, 0)

    pltpu.sync_copy(grid_v, out_hbm.at[ch, slab])


def kernel(coords, atom_types):
    r = jnp.asarray(_VDW)[atom_types]
    b = 1.5 * r
    minf = (coords - b[:, None]) / _GRID
    maxf = 2.0 + (coords + b[:, None]) / _GRID
    minp = jnp.clip(jnp.trunc(minf).astype(jnp.int32), 0, _N_GRID)
    maxp = jnp.clip(jnp.trunc(maxf).astype(jnp.int32), 0, _N_GRID)
    r2 = r * r
    c1 = -2.0 / r2
    a2 = 2.0 / (float(np.e) * r)

    pad = _NPAD - _N_ATOMS
    zpad_i = jnp.zeros((pad,), jnp.int32)

    def fbits(x):
        return jnp.concatenate(
            [lax.bitcast_convert_type(x, jnp.int32), zpad_i])

    def ipad(x):
        return jnp.concatenate([x, zpad_i])

    atoms = jnp.concatenate([
        jnp.concatenate([atom_types.astype(jnp.int32),
                         jnp.full((pad,), -1, jnp.int32)]),
        ipad(minp[:, 0]), ipad(maxp[:, 0]),
        ipad(minp[:, 1]), ipad(maxp[:, 1]),
        ipad(minp[:, 2]), ipad(maxp[:, 2]),
        fbits(coords[:, 0]), fbits(coords[:, 1]), fbits(coords[:, 2]),
        fbits(r2), fbits(c1), fbits(a2),
    ])

    mesh = plsc.VectorSubcoreMesh(core_axis_name="c", subcore_axis_name="s")
    f = functools.partial(
        pl.kernel,
        out_type=jax.ShapeDtypeStruct((8, 4, _GSZ), jnp.float32),
        mesh=mesh,
        scratch_types=[
            pltpu.VMEM((_NF * _NPAD,), jnp.int32),
            pltpu.VMEM((_GSZ,), jnp.float32),
            pltpu.VMEM((_WLSZ,), jnp.int32),
            pltpu.VMEM(((_SLAB + 2) * _L,), jnp.float32),
            pltpu.VMEM((_L * _L,), jnp.float32),
        ],
        compiler_params=pltpu.CompilerParams(needs_layout_passes=False),
    )(_sc_body)
    res = f(atoms)
    return res.reshape(8, 4, _SLAB, _N_GRID, _N_GRID).reshape(
        8, _N_GRID, _N_GRID, _N_GRID)
